# baseline (speedup 1.0000x reference)
import functools

import jax
import jax.numpy as jnp
from jax.experimental import pallas as pl
from jax.experimental.pallas import tpu as pltpu

HEADS = 4
CH = 128
HC = HEADS * CH
NEG_SLOPE = 0.2
K = 5
RECV_ID = 1
SAFE_ID = 2
BLK = 256
TS = 8
NEGBIG = -1e30
BIGI = 2 ** 30


def _window(b_t, bc_row):
    b_lo = jnp.min(b_t)
    b_hi = jnp.max(b_t)
    c_lo = jnp.sum((bc_row < b_lo).astype(jnp.int32))
    c_hi = jnp.sum((bc_row <= b_hi).astype(jnp.int32))
    j0 = c_lo // BLK
    j1 = (c_hi + BLK - 1) // BLK
    return j0, j1


def _knn_kernel(pxr, pyr, br, pxc, pyc, bc, nn_ref, val_ref):
    i = pl.program_id(0)
    t0 = i * BLK
    px_t = pxr[...]
    py_t = pyr[...]
    b_t = br[...]
    bc_row = bc[0:1, :]
    j0, j1 = _window(b_t, bc_row)

    tid = t0 + jax.lax.broadcasted_iota(jnp.int32, (BLK, 1), 0)
    iota_c = jax.lax.broadcasted_iota(jnp.int32, (BLK, BLK), 1)
    iota_m = jax.lax.broadcasted_iota(jnp.int32, (BLK, 2 * K), 1)

    def body(j, carry):
        bd, bi, nb = carry
        c0 = pl.multiple_of(j * BLK, BLK)
        px_s = pxc[0:1, pl.ds(c0, BLK)]
        py_s = pyc[0:1, pl.ds(c0, BLK)]
        b_s = bc[0:1, pl.ds(c0, BLK)]
        dx = px_t - px_s
        dy = py_t - py_s
        d = jnp.sqrt(dx * dx + dy * dy)
        sid = c0 + iota_c
        same = b_t == b_s
        dm = jnp.where(same & (tid != sid), d, jnp.inf)
        nb = nb + jnp.sum(same.astype(jnp.int32), axis=1, keepdims=True)
        cur = dm
        ld, li = [], []
        for _ in range(K):
            mv = jnp.min(cur, axis=1, keepdims=True)
            jj = jnp.min(jnp.where(cur == mv, iota_c, BIGI), axis=1,
                         keepdims=True)
            oh = iota_c == jj
            ld.append(mv)
            li.append(jj + c0)
            cur = jnp.where(oh, jnp.inf, cur)
        cd = jnp.concatenate([bd] + ld, axis=1)
        ci = jnp.concatenate([bi] + li, axis=1)
        cur = cd
        nd, ni = [], []
        for _ in range(K):
            mv = jnp.min(cur, axis=1, keepdims=True)
            jj = jnp.min(jnp.where(cur == mv, iota_m, BIGI), axis=1,
                         keepdims=True)
            oh = iota_m == jj
            nd.append(mv)
            ni.append(jnp.sum(jnp.where(oh, ci, 0), axis=1, keepdims=True))
            cur = jnp.where(oh, jnp.inf, cur)
        return (jnp.concatenate(nd, axis=1), jnp.concatenate(ni, axis=1), nb)

    bd0 = jnp.full((BLK, K), jnp.inf, jnp.float32)
    bi0 = jnp.zeros((BLK, K), jnp.int32)
    nb0 = jnp.zeros((BLK, 1), jnp.int32)
    _, bi, nb = jax.lax.fori_loop(j0, j1, body, (bd0, bi0, nb0))
    k_eff = jnp.clip(nb - 1, 0, K)
    slot = jax.lax.broadcasted_iota(jnp.int32, (BLK, K), 1)
    nn_ref[...] = bi
    val_ref[...] = (slot < k_eff).astype(jnp.int32)


def _mm_kernel(x, wl, wr, xl_ref, xr_ref):
    xv = x[...]
    xl_ref[...] = jnp.dot(xv, wl[...], preferred_element_type=jnp.float32)
    xr_ref[...] = jnp.dot(xv, wr[...], preferred_element_type=jnp.float32)


def _gat_kernel(xl, xr, nn_t, val_t, nnc, vc, br, rr, bc, rc, attf, attc,
                biasf, out_ref, lg_ref):
    i = pl.program_id(0)
    t0 = i * BLK
    b_t = br[...]
    bc_row = bc[0:1, :]
    j0, j1 = _window(b_t, bc_row)

    nn_b = nn_t[...]
    val_b = val_t[...]
    role_t = rr[...]
    tid = t0 + jax.lax.broadcasted_iota(jnp.int32, (BLK, 1), 0)
    iota_c = jax.lax.broadcasted_iota(jnp.int32, (1, BLK), 1)
    att_row = attf[0:1, :]
    la = 0.5 * (1.0 + NEG_SLOPE)
    lb = 0.5 * (1.0 - NEG_SLOPE)
    xr_t = xr[...]
    ar = [jnp.dot(xr_t[:, h * CH:(h + 1) * CH], attc[:, h:h + 1],
                  preferred_element_type=jnp.float32) for h in range(HEADS)]

    def body(j, carry):
        m, z, acc = carry
        c0 = pl.multiple_of(j * BLK, BLK)
        xl_s = xl[pl.ds(c0, BLK), :]
        sid = c0 + iota_c
        role_s = rc[0:1, pl.ds(c0, BLK)]
        a_dir = jnp.zeros((BLK, BLK), jnp.bool_)
        b_dir = jnp.zeros((BLK, BLK), jnp.bool_)
        for k in range(K):
            nnc_k = nnc[TS * k:TS * k + 1, pl.ds(c0, BLK)]
            vc_k = vc[TS * k:TS * k + 1, pl.ds(c0, BLK)]
            a_dir = a_dir | ((nnc_k == tid) & (vc_k > 0))
            b_dir = b_dir | ((nn_b[:, k:k + 1] == sid) & (val_b[:, k:k + 1] > 0))
        keep = jnp.logical_not((role_s == RECV_ID) & (role_t == SAFE_ID))
        mult = ((a_dir.astype(jnp.float32) + b_dir.astype(jnp.float32))
                * keep.astype(jnp.float32)
                + (tid == sid).astype(jnp.float32))

        al = [jax.lax.dot_general(
            attf[0:1, h * CH:(h + 1) * CH], xl_s[:, h * CH:(h + 1) * CH],
            (((1,), (1,)), ((), ())),
            preferred_element_type=jnp.float32) for h in range(HEADS)]

        def fill(ts, _):
            r0 = ts * TS
            xr8 = xr[pl.ds(r0, TS), :]
            w8 = jnp.abs(xr8[:, None, :] + xl_s[None, :, :]) * att_row[None, :, :]
            for h in range(HEADS):
                lg_ref[h, pl.ds(r0, TS), :] = jnp.sum(
                    w8[:, :, h * CH:(h + 1) * CH], axis=2)
            return 0

        jax.lax.fori_loop(0, BLK // TS, fill, 0)

        ms, zs, accs = [], [], []
        for h in range(HEADS):
            lg_h = la * (ar[h] + al[h]) + lb * lg_ref[h]
            lg_h = jnp.where(mult > 0, lg_h, NEGBIG)
            m_old = m[:, h:h + 1]
            m_new = jnp.maximum(m_old, jnp.max(lg_h, axis=1, keepdims=True))
            scale = jnp.exp(m_old - m_new)
            p = mult * jnp.exp(lg_h - m_new)
            zs.append(z[:, h:h + 1] * scale + jnp.sum(p, axis=1, keepdims=True))
            accs.append(acc[:, h * CH:(h + 1) * CH] * scale
                        + jnp.dot(p, xl_s[:, h * CH:(h + 1) * CH],
                                  preferred_element_type=jnp.float32))
            ms.append(m_new)
        return (jnp.concatenate(ms, axis=1), jnp.concatenate(zs, axis=1),
                jnp.concatenate(accs, axis=1))

    m0 = jnp.full((BLK, HEADS), NEGBIG, jnp.float32)
    z0 = jnp.zeros((BLK, HEADS), jnp.float32)
    a0 = jnp.zeros((BLK, HC), jnp.float32)
    m, z, acc = jax.lax.fori_loop(j0, j1, body, (m0, z0, a0))

    o = jnp.zeros((BLK, CH), jnp.float32)
    for h in range(HEADS):
        o = o + acc[:, h * CH:(h + 1) * CH] / (z[:, h:h + 1] + 1e-16)
    o = o * (1.0 / HEADS) + biasf[0:1, :]
    out_ref[...] = jnp.where(o > 0, o, jnp.exp(jnp.minimum(o, 0.0)) - 1.0)


def _full(shape):
    return pl.BlockSpec(shape, lambda i: tuple(0 for _ in shape))


def _rows(w):
    return pl.BlockSpec((BLK, w), lambda i: (i, 0))


@jax.jit
def kernel(x, pos, role_ids, batch, Wl0, Wr0, att0, b0, Wl1, Wr1, att1, b1,
           Wl2, Wr2, att2, b2):
    N = x.shape[0]
    D = x.shape[1]
    nblk = -(-N // BLK)
    NP = nblk * BLK
    padn = NP - N

    pos_p = jnp.pad(pos, ((0, padn), (0, 0)))
    batch_p = jnp.pad(batch, (0, padn), constant_values=1 << 20)
    role_p = jnp.pad(role_ids, (0, padn))
    x_p = jnp.pad(x, ((0, padn), (0, 0)))

    pxr = pos_p[:, 0:1]
    pyr = pos_p[:, 1:2]
    br = batch_p[:, None]
    rr = role_p[:, None]
    pxc = jnp.broadcast_to(pos_p[:, 0][None, :], (TS, NP))
    pyc = jnp.broadcast_to(pos_p[:, 1][None, :], (TS, NP))
    bc = jnp.broadcast_to(batch_p[None, :], (TS, NP))
    rc = jnp.broadcast_to(role_p[None, :], (TS, NP))

    nn, val = pl.pallas_call(
        _knn_kernel,
        grid=(nblk,),
        in_specs=[_rows(1), _rows(1), _rows(1),
                  _full((TS, NP)), _full((TS, NP)), _full((TS, NP))],
        out_specs=[_rows(K), _rows(K)],
        out_shape=[jax.ShapeDtypeStruct((NP, K), jnp.int32),
                   jax.ShapeDtypeStruct((NP, K), jnp.int32)],
    )(pxr, pyr, br, pxc, pyc, bc)

    nnc = jnp.repeat(nn.T, TS, axis=0)
    vc = jnp.repeat(val.T, TS, axis=0)

    mm = pl.pallas_call(
        _mm_kernel,
        grid=(nblk,),
        in_specs=[_rows(D), _full((D, HC)), _full((D, HC))],
        out_specs=[_rows(HC), _rows(HC)],
        out_shape=[jax.ShapeDtypeStruct((NP, HC), jnp.float32)] * 2,
    )

    gat = pl.pallas_call(
        _gat_kernel,
        grid=(nblk,),
        in_specs=[_full((NP, HC)), _rows(HC), _rows(K), _rows(K),
                  _full((K * TS, NP)), _full((K * TS, NP)),
                  _rows(1), _rows(1), _full((TS, NP)), _full((TS, NP)),
                  _full((TS, HC)), _full((CH, HEADS)), _full((TS, CH))],
        out_specs=_rows(CH),
        out_shape=jax.ShapeDtypeStruct((NP, CH), jnp.float32),
        scratch_shapes=[pltpu.VMEM((HEADS, BLK, BLK), jnp.float32)],
    )

    h = x_p
    for Wl, Wr, att, b in ((Wl0, Wr0, att0, b0), (Wl1, Wr1, att1, b1),
                           (Wl2, Wr2, att2, b2)):
        xl, xr = mm(h, Wl, Wr)
        attf = jnp.broadcast_to(att.reshape(1, HC), (TS, HC))
        attc = att.T
        biasf = jnp.broadcast_to(b[None, :], (TS, CH))
        h = gat(xl, xr, nn, val, nnc, vc, br, rr, bc, rc, attf, attc, biasf)
    return h[:N]


# speedup vs baseline: 1.6455x; 1.6455x over previous
import functools

import jax
import jax.numpy as jnp
from jax.experimental import pallas as pl
from jax.experimental.pallas import tpu as pltpu

HEADS = 4
CH = 128
HC = HEADS * CH
NEG_SLOPE = 0.2
K = 5
RECV_ID = 1
SAFE_ID = 2
BLK = 128
TS = 8
NEGBIG = -1e30
BIGI = 2 ** 30


def _window(b_t, bc_row):
    b_lo = jnp.min(b_t)
    b_hi = jnp.max(b_t)
    c_lo = jnp.sum((bc_row < b_lo).astype(jnp.int32))
    c_hi = jnp.sum((bc_row <= b_hi).astype(jnp.int32))
    j0 = c_lo // BLK
    j1 = (c_hi + BLK - 1) // BLK
    return j0, j1


def _knn_kernel(pxr, pyr, br, pxc, pyc, bc, nn_ref, val_ref):
    i = pl.program_id(0)
    t0 = i * BLK
    px_t = pxr[...]
    py_t = pyr[...]
    b_t = br[...]
    bc_row = bc[0:1, :]
    j0, j1 = _window(b_t, bc_row)

    tid = t0 + jax.lax.broadcasted_iota(jnp.int32, (BLK, 1), 0)
    iota_c = jax.lax.broadcasted_iota(jnp.int32, (BLK, BLK), 1)
    iota_m = jax.lax.broadcasted_iota(jnp.int32, (BLK, 2 * K), 1)

    def body(j, carry):
        bd, bi, nb = carry
        c0 = pl.multiple_of(j * BLK, BLK)
        px_s = pxc[0:1, pl.ds(c0, BLK)]
        py_s = pyc[0:1, pl.ds(c0, BLK)]
        b_s = bc[0:1, pl.ds(c0, BLK)]
        dx = px_t - px_s
        dy = py_t - py_s
        d = jnp.sqrt(dx * dx + dy * dy)
        sid = c0 + iota_c
        same = b_t == b_s
        dm = jnp.where(same & (tid != sid), d, jnp.inf)
        nb = nb + jnp.sum(same.astype(jnp.int32), axis=1, keepdims=True)
        cur = dm
        ld, li = [], []
        for _ in range(K):
            mv = jnp.min(cur, axis=1, keepdims=True)
            jj = jnp.min(jnp.where(cur == mv, iota_c, BIGI), axis=1,
                         keepdims=True)
            oh = iota_c == jj
            ld.append(mv)
            li.append(jj + c0)
            cur = jnp.where(oh, jnp.inf, cur)
        cd = jnp.concatenate([bd] + ld, axis=1)
        ci = jnp.concatenate([bi] + li, axis=1)
        cur = cd
        nd, ni = [], []
        for _ in range(K):
            mv = jnp.min(cur, axis=1, keepdims=True)
            jj = jnp.min(jnp.where(cur == mv, iota_m, BIGI), axis=1,
                         keepdims=True)
            oh = iota_m == jj
            nd.append(mv)
            ni.append(jnp.sum(jnp.where(oh, ci, 0), axis=1, keepdims=True))
            cur = jnp.where(oh, jnp.inf, cur)
        return (jnp.concatenate(nd, axis=1), jnp.concatenate(ni, axis=1), nb)

    bd0 = jnp.full((BLK, K), jnp.inf, jnp.float32)
    bi0 = jnp.zeros((BLK, K), jnp.int32)
    nb0 = jnp.zeros((BLK, 1), jnp.int32)
    _, bi, nb = jax.lax.fori_loop(j0, j1, body, (bd0, bi0, nb0))
    k_eff = jnp.clip(nb - 1, 0, K)
    slot = jax.lax.broadcasted_iota(jnp.int32, (BLK, K), 1)
    nn_ref[...] = bi
    val_ref[...] = (slot < k_eff).astype(jnp.int32)


def _mm_kernel(x, wl, wr, xl_ref, xr_ref):
    xv = x[...]
    xl_ref[...] = jnp.dot(xv, wl[...], preferred_element_type=jnp.float32)
    xr_ref[...] = jnp.dot(xv, wr[...], preferred_element_type=jnp.float32)


def _gat_kernel(xl, xr, nn_t, val_t, nnc, vc, br, rr, bc, rc, attf, attc,
                biasf, out_ref, lg_ref):
    i = pl.program_id(0)
    t0 = i * BLK
    b_t = br[...]
    bc_row = bc[0:1, :]
    j0, j1 = _window(b_t, bc_row)

    nn_b = nn_t[...]
    val_b = val_t[...]
    role_t = rr[...]
    tid = t0 + jax.lax.broadcasted_iota(jnp.int32, (BLK, 1), 0)
    iota_c = jax.lax.broadcasted_iota(jnp.int32, (1, BLK), 1)
    att_row = attf[0:1, :]
    la = 0.5 * (1.0 + NEG_SLOPE)
    lb = 0.5 * (1.0 - NEG_SLOPE)
    xr_t = xr[...]
    ar = [jnp.dot(xr_t[:, h * CH:(h + 1) * CH], attc[:, h:h + 1],
                  preferred_element_type=jnp.float32) for h in range(HEADS)]

    def body(j, carry):
        m, z, acc = carry
        c0 = pl.multiple_of(j * BLK, BLK)
        xl_s = xl[pl.ds(c0, BLK), :]
        sid = c0 + iota_c
        role_s = rc[0:1, pl.ds(c0, BLK)]
        a_dir = jnp.zeros((BLK, BLK), jnp.bool_)
        b_dir = jnp.zeros((BLK, BLK), jnp.bool_)
        for k in range(K):
            nnc_k = nnc[TS * k:TS * k + 1, pl.ds(c0, BLK)]
            vc_k = vc[TS * k:TS * k + 1, pl.ds(c0, BLK)]
            a_dir = a_dir | ((nnc_k == tid) & (vc_k > 0))
            b_dir = b_dir | ((nn_b[:, k:k + 1] == sid) & (val_b[:, k:k + 1] > 0))
        keep = jnp.logical_not((role_s == RECV_ID) & (role_t == SAFE_ID))
        mult = ((a_dir.astype(jnp.float32) + b_dir.astype(jnp.float32))
                * keep.astype(jnp.float32)
                + (tid == sid).astype(jnp.float32))

        al = [jax.lax.dot_general(
            attf[0:1, h * CH:(h + 1) * CH], xl_s[:, h * CH:(h + 1) * CH],
            (((1,), (1,)), ((), ())),
            preferred_element_type=jnp.float32) for h in range(HEADS)]

        def fill(ts, _):
            r0 = ts * TS
            xr8 = xr[pl.ds(r0, TS), :]
            w8 = jnp.abs(xr8[:, None, :] + xl_s[None, :, :]) * att_row[None, :, :]
            for h in range(HEADS):
                lg_ref[h, pl.ds(r0, TS), :] = jnp.sum(
                    w8[:, :, h * CH:(h + 1) * CH], axis=2)
            return 0

        jax.lax.fori_loop(0, BLK // TS, fill, 0)

        ms, zs, accs = [], [], []
        for h in range(HEADS):
            lg_h = la * (ar[h] + al[h]) + lb * lg_ref[h]
            lg_h = jnp.where(mult > 0, lg_h, NEGBIG)
            m_old = m[:, h:h + 1]
            m_new = jnp.maximum(m_old, jnp.max(lg_h, axis=1, keepdims=True))
            scale = jnp.exp(m_old - m_new)
            p = mult * jnp.exp(lg_h - m_new)
            zs.append(z[:, h:h + 1] * scale + jnp.sum(p, axis=1, keepdims=True))
            accs.append(acc[:, h * CH:(h + 1) * CH] * scale
                        + jnp.dot(p, xl_s[:, h * CH:(h + 1) * CH],
                                  preferred_element_type=jnp.float32))
            ms.append(m_new)
        return (jnp.concatenate(ms, axis=1), jnp.concatenate(zs, axis=1),
                jnp.concatenate(accs, axis=1))

    m0 = jnp.full((BLK, HEADS), NEGBIG, jnp.float32)
    z0 = jnp.zeros((BLK, HEADS), jnp.float32)
    a0 = jnp.zeros((BLK, HC), jnp.float32)
    m, z, acc = jax.lax.fori_loop(j0, j1, body, (m0, z0, a0))

    o = jnp.zeros((BLK, CH), jnp.float32)
    for h in range(HEADS):
        o = o + acc[:, h * CH:(h + 1) * CH] / (z[:, h:h + 1] + 1e-16)
    o = o * (1.0 / HEADS) + biasf[0:1, :]
    out_ref[...] = jnp.where(o > 0, o, jnp.exp(jnp.minimum(o, 0.0)) - 1.0)


def _full(shape):
    return pl.BlockSpec(shape, lambda i: tuple(0 for _ in shape))


def _rows(w):
    return pl.BlockSpec((BLK, w), lambda i: (i, 0))


@jax.jit
def kernel(x, pos, role_ids, batch, Wl0, Wr0, att0, b0, Wl1, Wr1, att1, b1,
           Wl2, Wr2, att2, b2):
    N = x.shape[0]
    D = x.shape[1]
    nblk = -(-N // BLK)
    NP = nblk * BLK
    padn = NP - N

    pos_p = jnp.pad(pos, ((0, padn), (0, 0)))
    batch_p = jnp.pad(batch, (0, padn), constant_values=1 << 20)
    role_p = jnp.pad(role_ids, (0, padn))
    x_p = jnp.pad(x, ((0, padn), (0, 0)))

    pxr = pos_p[:, 0:1]
    pyr = pos_p[:, 1:2]
    br = batch_p[:, None]
    rr = role_p[:, None]
    pxc = jnp.broadcast_to(pos_p[:, 0][None, :], (TS, NP))
    pyc = jnp.broadcast_to(pos_p[:, 1][None, :], (TS, NP))
    bc = jnp.broadcast_to(batch_p[None, :], (TS, NP))
    rc = jnp.broadcast_to(role_p[None, :], (TS, NP))

    nn, val = pl.pallas_call(
        _knn_kernel,
        grid=(nblk,),
        in_specs=[_rows(1), _rows(1), _rows(1),
                  _full((TS, NP)), _full((TS, NP)), _full((TS, NP))],
        out_specs=[_rows(K), _rows(K)],
        out_shape=[jax.ShapeDtypeStruct((NP, K), jnp.int32),
                   jax.ShapeDtypeStruct((NP, K), jnp.int32)],
    )(pxr, pyr, br, pxc, pyc, bc)

    nnc = jnp.repeat(nn.T, TS, axis=0)
    vc = jnp.repeat(val.T, TS, axis=0)

    mm = pl.pallas_call(
        _mm_kernel,
        grid=(nblk,),
        in_specs=[_rows(D), _full((D, HC)), _full((D, HC))],
        out_specs=[_rows(HC), _rows(HC)],
        out_shape=[jax.ShapeDtypeStruct((NP, HC), jnp.float32)] * 2,
    )

    gat = pl.pallas_call(
        _gat_kernel,
        grid=(nblk,),
        in_specs=[_full((NP, HC)), _rows(HC), _rows(K), _rows(K),
                  _full((K * TS, NP)), _full((K * TS, NP)),
                  _rows(1), _rows(1), _full((TS, NP)), _full((TS, NP)),
                  _full((TS, HC)), _full((CH, HEADS)), _full((TS, CH))],
        out_specs=_rows(CH),
        out_shape=jax.ShapeDtypeStruct((NP, CH), jnp.float32),
        scratch_shapes=[pltpu.VMEM((HEADS, BLK, BLK), jnp.float32)],
    )

    h = x_p
    for Wl, Wr, att, b in ((Wl0, Wr0, att0, b0), (Wl1, Wr1, att1, b1),
                           (Wl2, Wr2, att2, b2)):
        xl, xr = mm(h, Wl, Wr)
        attf = jnp.broadcast_to(att.reshape(1, HC), (TS, HC))
        attc = att.T
        biasf = jnp.broadcast_to(b[None, :], (TS, CH))
        h = gat(xl, xr, nn, val, nnc, vc, br, rr, bc, rc, attf, attc, biasf)
    return h[:N]


# speedup vs baseline: 1.7392x; 1.0570x over previous
import functools

import jax
import jax.numpy as jnp
from jax.experimental import pallas as pl
from jax.experimental.pallas import tpu as pltpu

HEADS = 4
CH = 128
HC = HEADS * CH
NEG_SLOPE = 0.2
K = 5
RECV_ID = 1
SAFE_ID = 2
BLK = 128
TS = 16
NEGBIG = -1e30
BIGI = 2 ** 30


def _window(b_t, bc_row):
    b_lo = jnp.min(b_t)
    b_hi = jnp.max(b_t)
    c_lo = jnp.sum((bc_row < b_lo).astype(jnp.int32))
    c_hi = jnp.sum((bc_row <= b_hi).astype(jnp.int32))
    j0 = c_lo // BLK
    j1 = (c_hi + BLK - 1) // BLK
    return j0, j1


def _knn_kernel(pxr, pyr, br, pxc, pyc, bc, nn_ref, val_ref):
    i = pl.program_id(0)
    t0 = i * BLK
    px_t = pxr[...]
    py_t = pyr[...]
    b_t = br[...]
    bc_row = bc[0:1, :]
    j0, j1 = _window(b_t, bc_row)

    tid = t0 + jax.lax.broadcasted_iota(jnp.int32, (BLK, 1), 0)
    iota_c = jax.lax.broadcasted_iota(jnp.int32, (BLK, BLK), 1)
    iota_m = jax.lax.broadcasted_iota(jnp.int32, (BLK, 2 * K), 1)

    def body(j, carry):
        bd, bi, nb = carry
        c0 = pl.multiple_of(j * BLK, BLK)
        px_s = pxc[0:1, pl.ds(c0, BLK)]
        py_s = pyc[0:1, pl.ds(c0, BLK)]
        b_s = bc[0:1, pl.ds(c0, BLK)]
        dx = px_t - px_s
        dy = py_t - py_s
        d = jnp.sqrt(dx * dx + dy * dy)
        sid = c0 + iota_c
        same = b_t == b_s
        dm = jnp.where(same & (tid != sid), d, jnp.inf)
        nb = nb + jnp.sum(same.astype(jnp.int32), axis=1, keepdims=True)
        cur = dm
        ld, li = [], []
        for _ in range(K):
            mv = jnp.min(cur, axis=1, keepdims=True)
            jj = jnp.min(jnp.where(cur == mv, iota_c, BIGI), axis=1,
                         keepdims=True)
            oh = iota_c == jj
            ld.append(mv)
            li.append(jj + c0)
            cur = jnp.where(oh, jnp.inf, cur)
        cd = jnp.concatenate([bd] + ld, axis=1)
        ci = jnp.concatenate([bi] + li, axis=1)
        cur = cd
        nd, ni = [], []
        for _ in range(K):
            mv = jnp.min(cur, axis=1, keepdims=True)
            jj = jnp.min(jnp.where(cur == mv, iota_m, BIGI), axis=1,
                         keepdims=True)
            oh = iota_m == jj
            nd.append(mv)
            ni.append(jnp.sum(jnp.where(oh, ci, 0), axis=1, keepdims=True))
            cur = jnp.where(oh, jnp.inf, cur)
        return (jnp.concatenate(nd, axis=1), jnp.concatenate(ni, axis=1), nb)

    bd0 = jnp.full((BLK, K), jnp.inf, jnp.float32)
    bi0 = jnp.zeros((BLK, K), jnp.int32)
    nb0 = jnp.zeros((BLK, 1), jnp.int32)
    _, bi, nb = jax.lax.fori_loop(j0, j1, body, (bd0, bi0, nb0))
    k_eff = jnp.clip(nb - 1, 0, K)
    slot = jax.lax.broadcasted_iota(jnp.int32, (BLK, K), 1)
    nn_ref[...] = bi
    val_ref[...] = (slot < k_eff).astype(jnp.int32)


def _mm_kernel(x, wl, wr, xl_ref, xr_ref):
    xv = x[...]
    xl_ref[...] = jnp.dot(xv, wl[...], preferred_element_type=jnp.float32)
    xr_ref[...] = jnp.dot(xv, wr[...], preferred_element_type=jnp.float32)


def _gat_kernel(xl, xr, nn_t, val_t, nnc, vc, br, rr, bc, rc, attf, attc,
                biasf, out_ref, lg_ref):
    i = pl.program_id(0)
    t0 = i * BLK
    b_t = br[...]
    bc_row = bc[0:1, :]
    j0, j1 = _window(b_t, bc_row)

    nn_b = nn_t[...]
    val_b = val_t[...]
    role_t = rr[...]
    tid = t0 + jax.lax.broadcasted_iota(jnp.int32, (BLK, 1), 0)
    iota_c = jax.lax.broadcasted_iota(jnp.int32, (1, BLK), 1)
    att_row = attf[0:1, :]
    la = 0.5 * (1.0 + NEG_SLOPE)
    lb = 0.5 * (1.0 - NEG_SLOPE)
    xr_t = xr[...]
    ar = [jnp.dot(xr_t[:, h * CH:(h + 1) * CH], attc[:, h:h + 1],
                  preferred_element_type=jnp.float32) for h in range(HEADS)]

    def body(j, carry):
        m, z, acc = carry
        c0 = pl.multiple_of(j * BLK, BLK)
        xl_s = xl[pl.ds(c0, BLK), :]
        sid = c0 + iota_c
        role_s = rc[0:1, pl.ds(c0, BLK)]
        a_dir = jnp.zeros((BLK, BLK), jnp.bool_)
        b_dir = jnp.zeros((BLK, BLK), jnp.bool_)
        for k in range(K):
            nnc_k = nnc[TS * k:TS * k + 1, pl.ds(c0, BLK)]
            vc_k = vc[TS * k:TS * k + 1, pl.ds(c0, BLK)]
            a_dir = a_dir | ((nnc_k == tid) & (vc_k > 0))
            b_dir = b_dir | ((nn_b[:, k:k + 1] == sid) & (val_b[:, k:k + 1] > 0))
        keep = jnp.logical_not((role_s == RECV_ID) & (role_t == SAFE_ID))
        mult = ((a_dir.astype(jnp.float32) + b_dir.astype(jnp.float32))
                * keep.astype(jnp.float32)
                + (tid == sid).astype(jnp.float32))

        al = [jax.lax.dot_general(
            attf[0:1, h * CH:(h + 1) * CH], xl_s[:, h * CH:(h + 1) * CH],
            (((1,), (1,)), ((), ())),
            preferred_element_type=jnp.float32) for h in range(HEADS)]

        def fill(ts, _):
            r0 = ts * TS
            xr8 = xr[pl.ds(r0, TS), :]
            w8 = jnp.abs(xr8[:, None, :] + xl_s[None, :, :]) * att_row[None, :, :]
            for h in range(HEADS):
                lg_ref[h, pl.ds(r0, TS), :] = jnp.sum(
                    w8[:, :, h * CH:(h + 1) * CH], axis=2)
            return 0

        jax.lax.fori_loop(0, BLK // TS, fill, 0)

        ms, zs, accs = [], [], []
        for h in range(HEADS):
            lg_h = la * (ar[h] + al[h]) + lb * lg_ref[h]
            lg_h = jnp.where(mult > 0, lg_h, NEGBIG)
            m_old = m[:, h:h + 1]
            m_new = jnp.maximum(m_old, jnp.max(lg_h, axis=1, keepdims=True))
            scale = jnp.exp(m_old - m_new)
            p = mult * jnp.exp(lg_h - m_new)
            zs.append(z[:, h:h + 1] * scale + jnp.sum(p, axis=1, keepdims=True))
            accs.append(acc[:, h * CH:(h + 1) * CH] * scale
                        + jnp.dot(p, xl_s[:, h * CH:(h + 1) * CH],
                                  preferred_element_type=jnp.float32))
            ms.append(m_new)
        return (jnp.concatenate(ms, axis=1), jnp.concatenate(zs, axis=1),
                jnp.concatenate(accs, axis=1))

    m0 = jnp.full((BLK, HEADS), NEGBIG, jnp.float32)
    z0 = jnp.zeros((BLK, HEADS), jnp.float32)
    a0 = jnp.zeros((BLK, HC), jnp.float32)
    m, z, acc = jax.lax.fori_loop(j0, j1, body, (m0, z0, a0))

    o = jnp.zeros((BLK, CH), jnp.float32)
    for h in range(HEADS):
        o = o + acc[:, h * CH:(h + 1) * CH] / (z[:, h:h + 1] + 1e-16)
    o = o * (1.0 / HEADS) + biasf[0:1, :]
    out_ref[...] = jnp.where(o > 0, o, jnp.exp(jnp.minimum(o, 0.0)) - 1.0)


def _full(shape):
    return pl.BlockSpec(shape, lambda i: tuple(0 for _ in shape))


def _rows(w):
    return pl.BlockSpec((BLK, w), lambda i: (i, 0))


@jax.jit
def kernel(x, pos, role_ids, batch, Wl0, Wr0, att0, b0, Wl1, Wr1, att1, b1,
           Wl2, Wr2, att2, b2):
    N = x.shape[0]
    D = x.shape[1]
    nblk = -(-N // BLK)
    NP = nblk * BLK
    padn = NP - N

    pos_p = jnp.pad(pos, ((0, padn), (0, 0)))
    batch_p = jnp.pad(batch, (0, padn), constant_values=1 << 20)
    role_p = jnp.pad(role_ids, (0, padn))
    x_p = jnp.pad(x, ((0, padn), (0, 0)))

    pxr = pos_p[:, 0:1]
    pyr = pos_p[:, 1:2]
    br = batch_p[:, None]
    rr = role_p[:, None]
    pxc = jnp.broadcast_to(pos_p[:, 0][None, :], (TS, NP))
    pyc = jnp.broadcast_to(pos_p[:, 1][None, :], (TS, NP))
    bc = jnp.broadcast_to(batch_p[None, :], (TS, NP))
    rc = jnp.broadcast_to(role_p[None, :], (TS, NP))

    nn, val = pl.pallas_call(
        _knn_kernel,
        grid=(nblk,),
        in_specs=[_rows(1), _rows(1), _rows(1),
                  _full((TS, NP)), _full((TS, NP)), _full((TS, NP))],
        out_specs=[_rows(K), _rows(K)],
        out_shape=[jax.ShapeDtypeStruct((NP, K), jnp.int32),
                   jax.ShapeDtypeStruct((NP, K), jnp.int32)],
    )(pxr, pyr, br, pxc, pyc, bc)

    nnc = jnp.repeat(nn.T, TS, axis=0)
    vc = jnp.repeat(val.T, TS, axis=0)

    mm = pl.pallas_call(
        _mm_kernel,
        grid=(nblk,),
        in_specs=[_rows(D), _full((D, HC)), _full((D, HC))],
        out_specs=[_rows(HC), _rows(HC)],
        out_shape=[jax.ShapeDtypeStruct((NP, HC), jnp.float32)] * 2,
    )

    gat = pl.pallas_call(
        _gat_kernel,
        grid=(nblk,),
        in_specs=[_full((NP, HC)), _rows(HC), _rows(K), _rows(K),
                  _full((K * TS, NP)), _full((K * TS, NP)),
                  _rows(1), _rows(1), _full((TS, NP)), _full((TS, NP)),
                  _full((TS, HC)), _full((CH, HEADS)), _full((TS, CH))],
        out_specs=_rows(CH),
        out_shape=jax.ShapeDtypeStruct((NP, CH), jnp.float32),
        scratch_shapes=[pltpu.VMEM((HEADS, BLK, BLK), jnp.float32)],
    )

    h = x_p
    for Wl, Wr, att, b in ((Wl0, Wr0, att0, b0), (Wl1, Wr1, att1, b1),
                           (Wl2, Wr2, att2, b2)):
        xl, xr = mm(h, Wl, Wr)
        attf = jnp.broadcast_to(att.reshape(1, HC), (TS, HC))
        attc = att.T
        biasf = jnp.broadcast_to(b[None, :], (TS, CH))
        h = gat(xl, xr, nn, val, nnc, vc, br, rr, bc, rc, attf, attc, biasf)
    return h[:N]
